# baseline (speedup 1.0000x reference)
import functools

import jax
import jax.numpy as jnp
from jax import lax
from jax.experimental import pallas as pl
from jax.experimental.pallas import tpu as pltpu
from jax.experimental.pallas import tpu_sc as plsc

_EMB_DIM = 64
_PAD_DIM = 128
_BATCH = 16384
_VOCAB = 100000
_NBLK = (_VOCAB + _PAD_DIM - 1) // _PAD_DIM
_VPAD = _NBLK * _PAD_DIM

_NC = 2
_NS = 16
_NW = _NC * _NS
_BLK_PER_W = (_NBLK + _NW - 1) // _NW
_B_PER_W = _BATCH // _NW
_CHUNK = 128
_N_CHUNKS = _B_PER_W // _CHUNK

_mesh = plsc.VectorSubcoreMesh(core_axis_name="c", subcore_axis_name="s")


def _wid():
    return lax.axis_index("s") * _NC + lax.axis_index("c")


@functools.partial(
    pl.kernel,
    mesh=_mesh,
    out_type=jax.ShapeDtypeStruct((_VPAD, _PAD_DIM), jnp.float32),
    scratch_types=[
        pltpu.VMEM((_EMB_DIM, _PAD_DIM), jnp.float32),
        pltpu.VMEM((_PAD_DIM, _PAD_DIM), jnp.float32),
    ],
    compiler_params=pltpu.CompilerParams(
        use_tc_tiling_on_sc=True, needs_layout_passes=False
    ),
)
def _sc_relayout(tabt_hbm, out_hbm, in_v, out_v):
    wid = _wid()
    grows = [
        jnp.arange(16 * g, 16 * g + 16, dtype=jnp.int32)
        for g in range(_EMB_DIM // 16)
    ]
    zeros16 = jnp.zeros((16,), jnp.int32)

    def do_block(k, carry):
        c = k * _NW + wid

        @pl.when(c < _NBLK)
        def _():
            col = pl.multiple_of(c * _PAD_DIM, _PAD_DIM)
            pltpu.sync_copy(tabt_hbm.at[:, pl.ds(col, _PAD_DIM)], in_v)

            def do_row(l, carry2):
                lvec = zeros16 + l
                for g in range(_EMB_DIM // 16):
                    vals = plsc.load_gather(in_v, [grows[g], lvec])
                    out_v[l, pl.ds(16 * g, 16)] = vals
                return carry2

            lax.fori_loop(0, _PAD_DIM, do_row, 0, unroll=2)
            pltpu.sync_copy(out_v, out_hbm.at[pl.ds(col, _PAD_DIM)])

        return carry

    lax.fori_loop(0, _BLK_PER_W, do_block, 0)


@functools.partial(
    pl.kernel,
    mesh=_mesh,
    out_type=jax.ShapeDtypeStruct((_BATCH, _PAD_DIM), jnp.float32),
    scratch_types=[
        pltpu.VMEM((_N_CHUNKS, _CHUNK), jnp.int32),
        pltpu.VMEM((_B_PER_W, _PAD_DIM), jnp.float32),
        pltpu.SemaphoreType.DMA,
    ],
    compiler_params=pltpu.CompilerParams(use_tc_tiling_on_sc=False),
)
def _sc_gather(table_hbm, idx_hbm, out_hbm, idx_v, rows_v, sem):
    wid = _wid()
    pltpu.sync_copy(idx_hbm.at[wid], idx_v)
    copies = []
    for j in range(_N_CHUNKS):
        copies.append(
            pltpu.async_copy(
                table_hbm.at[idx_v.at[j]],
                rows_v.at[pl.ds(j * _CHUNK, _CHUNK)],
                sem,
            )
        )
    for c in copies:
        c.wait()
    pltpu.sync_copy(rows_v, out_hbm.at[pl.ds(wid * _B_PER_W, _B_PER_W)])


def kernel(terms, go_table):
    staged = _sc_relayout(go_table.T)
    idx = terms.astype(jnp.int32).reshape(_NW, _N_CHUNKS, _CHUNK)
    out = _sc_gather(staged, idx)
    return out[:, :_EMB_DIM]


# speedup vs baseline: 1.2103x; 1.2103x over previous
import functools

import jax
import jax.numpy as jnp
from jax import lax
from jax.experimental import pallas as pl
from jax.experimental.pallas import tpu as pltpu
from jax.experimental.pallas import tpu_sc as plsc

_EMB_DIM = 64
_PAD_DIM = 128
_BATCH = 16384
_VOCAB = 100000
_NBLK = (_VOCAB + _PAD_DIM - 1) // _PAD_DIM
_VPAD = _NBLK * _PAD_DIM

_NC = 2
_NS = 16
_NW = _NC * _NS
_BLK_PER_W = (_NBLK + _NW - 1) // _NW
_B_PER_W = _BATCH // _NW
_CHUNK = 128
_N_CHUNKS = _B_PER_W // _CHUNK

_mesh = plsc.VectorSubcoreMesh(core_axis_name="c", subcore_axis_name="s")


def _wid():
    return lax.axis_index("s") * _NC + lax.axis_index("c")


@functools.partial(
    pl.kernel,
    mesh=_mesh,
    out_type=jax.ShapeDtypeStruct((_VPAD, _PAD_DIM), jnp.float32),
    scratch_types=[
        pltpu.VMEM((_EMB_DIM, _PAD_DIM), jnp.float32),
        pltpu.VMEM((_EMB_DIM, _PAD_DIM), jnp.float32),
        pltpu.VMEM((_PAD_DIM, _PAD_DIM), jnp.float32),
        pltpu.VMEM((_PAD_DIM, _PAD_DIM), jnp.float32),
        pltpu.SemaphoreType.DMA,
        pltpu.SemaphoreType.DMA,
        pltpu.SemaphoreType.DMA,
        pltpu.SemaphoreType.DMA,
    ],
    compiler_params=pltpu.CompilerParams(
        use_tc_tiling_on_sc=True, needs_layout_passes=False
    ),
)
def _sc_relayout(tabt_hbm, out_hbm, iv0, iv1, ov0, ov1, si0, si1, so0, so1):
    wid = _wid()
    in_v = (iv0, iv1)
    out_v = (ov0, ov1)
    sem_in = (si0, si1)
    sem_out = (so0, so1)
    grows = [
        jnp.arange(16 * g, 16 * g + 16, dtype=jnp.int32)
        for g in range(_EMB_DIM // 16)
    ]
    zeros16 = jnp.zeros((16,), jnp.int32)

    last_ok = wid < (_NBLK - (_BLK_PER_W - 1) * _NW)

    def col_of(k):
        return pl.multiple_of(
            k * _NW * _PAD_DIM + wid * _PAD_DIM, _PAD_DIM
        )

    def start_in(k, p):
        return pltpu.async_copy(
            tabt_hbm.at[:, pl.ds(col_of(k), _PAD_DIM)],
            in_v[p],
            sem_in[p],
        )

    def transpose(p):
        src, dst = in_v[p], out_v[p]

        def do_row(l, carry):
            lvec = zeros16 + l
            for g in range(_EMB_DIM // 16):
                dst[l, pl.ds(16 * g, 16)] = plsc.load_gather(src, [grows[g], lvec])
            return carry

        lax.fori_loop(0, _PAD_DIM, do_row, 0, unroll=8)

    def start_out(k, p):
        return pltpu.async_copy(
            out_v[p],
            out_hbm.at[pl.ds(col_of(k), _PAD_DIM)],
            sem_out[p],
        )

    def wait_in(p):
        pltpu.make_async_copy(
            tabt_hbm.at[:, pl.ds(0, _PAD_DIM)], in_v[p], sem_in[p]
        ).wait()

    def wait_out(p):
        pltpu.make_async_copy(
            out_v[p], out_hbm.at[pl.ds(0, _PAD_DIM)], sem_out[p]
        ).wait()

    npair = (_BLK_PER_W - 1) // 2
    start_in(0, 0)

    def pair_body(j, carry):
        a = 2 * j
        wait_in(0)
        start_in(a + 1, 1)

        @pl.when(j > 0)
        def _():
            wait_out(0)

        transpose(0)
        start_out(a, 0)
        wait_in(1)
        nxt = a + 2

        @pl.when((nxt < 2 * npair) | ((nxt == 2 * npair) & last_ok))
        def _():
            start_in(nxt, 0)

        @pl.when(j > 0)
        def _():
            wait_out(1)

        transpose(1)
        start_out(a + 1, 1)
        return carry

    lax.fori_loop(0, npair, pair_body, 0)
    wait_out(0)

    @pl.when(last_ok)
    def _():
        wait_in(0)
        transpose(0)
        start_out(2 * npair, 0).wait()

    wait_out(1)


@functools.partial(
    pl.kernel,
    mesh=_mesh,
    out_type=jax.ShapeDtypeStruct((_BATCH, _PAD_DIM), jnp.float32),
    scratch_types=[
        pltpu.VMEM((_N_CHUNKS, _CHUNK), jnp.int32),
        pltpu.VMEM((_B_PER_W, _PAD_DIM), jnp.float32),
        pltpu.SemaphoreType.DMA,
    ],
    compiler_params=pltpu.CompilerParams(use_tc_tiling_on_sc=False),
)
def _sc_gather(table_hbm, idx_hbm, out_hbm, idx_v, rows_v, sem):
    wid = _wid()
    pltpu.sync_copy(idx_hbm.at[wid], idx_v)
    copies = []
    for j in range(_N_CHUNKS):
        copies.append(
            pltpu.async_copy(
                table_hbm.at[idx_v.at[j]],
                rows_v.at[pl.ds(j * _CHUNK, _CHUNK)],
                sem,
            )
        )
    for c in copies:
        c.wait()
    pltpu.sync_copy(rows_v, out_hbm.at[pl.ds(wid * _B_PER_W, _B_PER_W)])


def kernel(terms, go_table):
    staged = _sc_relayout(go_table.T)
    idx = terms.astype(jnp.int32).reshape(_NW, _N_CHUNKS, _CHUNK)
    out = _sc_gather(staged, idx)
    return out[:, :_EMB_DIM]


# speedup vs baseline: 1.9632x; 1.6221x over previous
import functools

import jax
import jax.numpy as jnp
from jax import lax
from jax.experimental import pallas as pl
from jax.experimental.pallas import tpu as pltpu
from jax.experimental.pallas import tpu_sc as plsc

_EMB_DIM = 64
_PAD_DIM = 128
_BATCH = 16384
_VOCAB = 100000

_NC = 2
_NS = 16
_NW = _NC * _NS
_B_PER_W = _BATCH // _NW
_CHUNK = 128
_N_CHUNKS = _B_PER_W // _CHUNK

_mesh = plsc.VectorSubcoreMesh(core_axis_name="c", subcore_axis_name="s")


@functools.partial(
    pl.kernel,
    mesh=_mesh,
    out_type=jax.ShapeDtypeStruct((_BATCH, _PAD_DIM), jnp.float32),
    scratch_types=[
        pltpu.VMEM((_N_CHUNKS, _CHUNK), jnp.int32),
        pltpu.VMEM((_B_PER_W, _PAD_DIM), jnp.float32),
        pltpu.SemaphoreType.DMA,
    ],
    compiler_params=pltpu.CompilerParams(use_tc_tiling_on_sc=False),
)
def _sc_gather(table_hbm, idx_hbm, out_hbm, idx_v, rows_v, sem):
    wid = lax.axis_index("s") * _NC + lax.axis_index("c")
    pltpu.sync_copy(idx_hbm.at[wid], idx_v)
    copies = []
    for j in range(_N_CHUNKS):
        copies.append(
            pltpu.async_copy(
                table_hbm.at[idx_v.at[j]],
                rows_v.at[pl.ds(j * _CHUNK, _CHUNK)],
                sem,
            )
        )
    for c in copies:
        c.wait()
    pltpu.sync_copy(rows_v, out_hbm.at[pl.ds(wid * _B_PER_W, _B_PER_W)])


def kernel(terms, go_table):
    tpad = jnp.zeros((_VOCAB, _PAD_DIM), jnp.float32).at[:, :_EMB_DIM].set(go_table)
    idx = terms.astype(jnp.int32).reshape(_NW, _N_CHUNKS, _CHUNK)
    out = _sc_gather(tpad, idx)
    return out[:, :_EMB_DIM]


# speedup vs baseline: 2.1361x; 1.0881x over previous
import functools

import jax
import jax.numpy as jnp
from jax import lax
from jax.experimental import pallas as pl
from jax.experimental.pallas import tpu as pltpu
from jax.experimental.pallas import tpu_sc as plsc

_EMB_DIM = 64
_PAD_DIM = 128
_BATCH = 16384
_VOCAB = 100000
_GRP = 8
_GRP_W = _GRP * _EMB_DIM
_NGRP = _VOCAB // _GRP

_NC = 2
_NS = 16
_NW = _NC * _NS
_B_PER_W = _BATCH // _NW
_CH = 64
_N_CH = _B_PER_W // _CH

_mesh = plsc.VectorSubcoreMesh(core_axis_name="c", subcore_axis_name="s")


@functools.partial(
    pl.kernel,
    mesh=_mesh,
    out_type=jax.ShapeDtypeStruct((_BATCH, _PAD_DIM), jnp.float32),
    scratch_types=[
        pltpu.VMEM((_N_CH, _CH), jnp.int32),
        pltpu.VMEM((_N_CH, _CH), jnp.int32),
        pltpu.VMEM((_CH, _GRP_W), jnp.float32),
        pltpu.VMEM((_CH, _GRP_W), jnp.float32),
        pltpu.VMEM((_CH, _PAD_DIM), jnp.float32),
        pltpu.VMEM((_CH, _PAD_DIM), jnp.float32),
        pltpu.SemaphoreType.DMA,
        pltpu.SemaphoreType.DMA,
        pltpu.SemaphoreType.DMA,
        pltpu.SemaphoreType.DMA,
    ],
    compiler_params=pltpu.CompilerParams(
        use_tc_tiling_on_sc=False, needs_layout_passes=False
    ),
)
def _sc_group_gather(
    tab_hbm, gid_hbm, rof_hbm, out_hbm,
    gid_v, rof_v, tb0, tb1, sb0, sb1, sg0, sg1, so0, so1,
):
    wid = lax.axis_index("s") * _NC + lax.axis_index("c")
    tb = (tb0, tb1)
    sb = (sb0, sb1)
    sem_g = (sg0, sg1)
    sem_o = (so0, so1)
    pltpu.sync_copy(gid_hbm.at[wid], gid_v)
    pltpu.sync_copy(rof_hbm.at[wid], rof_v)
    wbase = wid * _B_PER_W
    lane = jnp.arange(16, dtype=jnp.int32)

    def start_chunk(k, p):
        return pltpu.async_copy(tab_hbm.at[gid_v.at[k]], tb[p], sem_g[p])

    def wait_chunk(p):
        pltpu.make_async_copy(tab_hbm.at[gid_v.at[0]], tb[p], sem_g[p]).wait()

    def extract(k, p):
        for q in range(_CH // 16):
            rvec = rof_v[k, pl.ds(16 * q, 16)] * _EMB_DIM
            for j in range(16):
                roff = jnp.sum(jnp.where(lane == j, rvec, 0))
                jj = 16 * q + j
                for g in range(_EMB_DIM // 16):
                    sb[p][jj, pl.ds(16 * g, 16)] = tb[p][
                        jj, pl.ds(roff + 16 * g, 16)
                    ]

    def start_out(k, p):
        return pltpu.async_copy(
            sb[p], out_hbm.at[pl.ds(wbase + k * _CH, _CH)], sem_o[p]
        )

    def wait_out(p):
        pltpu.make_async_copy(
            sb[p], out_hbm.at[pl.ds(0, _CH)], sem_o[p]
        ).wait()

    npair = _N_CH // 2
    start_chunk(0, 0)

    def pair_body(i, carry):
        a = 2 * i
        start_chunk(a + 1, 1)
        wait_chunk(0)

        @pl.when(i > 0)
        def _():
            wait_out(0)

        extract(a, 0)
        start_out(a, 0)

        @pl.when(i < npair - 1)
        def _():
            start_chunk(a + 2, 0)

        wait_chunk(1)

        @pl.when(i > 0)
        def _():
            wait_out(1)

        extract(a + 1, 1)
        start_out(a + 1, 1)
        return carry

    lax.fori_loop(0, npair, pair_body, 0)
    wait_out(0)
    wait_out(1)


def kernel(terms, go_table):
    tab2 = go_table.reshape(_NGRP, _GRP_W)
    idx = terms.astype(jnp.int32)
    gid = (idx >> 3).reshape(_NW, _N_CH, _CH)
    rof = (idx & 7).reshape(_NW, _N_CH, _CH)
    out = _sc_group_gather(tab2, gid, rof)
    return out[:, :_EMB_DIM]


# speedup vs baseline: 2.7302x; 1.2781x over previous
import functools

import jax
import jax.numpy as jnp
from jax import lax
from jax.experimental import pallas as pl
from jax.experimental.pallas import tpu as pltpu
from jax.experimental.pallas import tpu_sc as plsc

_EMB_DIM = 64
_PAD_DIM = 128
_BATCH = 16384

_NC = 2
_NS = 16
_NW = _NC * _NS
_B_PER_W = _BATCH // _NW
_CHUNK = 128
_N_CHUNKS = _B_PER_W // _CHUNK

_mesh = plsc.VectorSubcoreMesh(core_axis_name="c", subcore_axis_name="s")


@functools.partial(
    pl.kernel,
    mesh=_mesh,
    out_type=jax.ShapeDtypeStruct((_BATCH, _PAD_DIM), jnp.float32),
    scratch_types=[
        pltpu.VMEM((_N_CHUNKS, _CHUNK), jnp.int32),
        pltpu.VMEM((_B_PER_W, _PAD_DIM), jnp.float32),
        pltpu.SemaphoreType.DMA,
    ],
    compiler_params=pltpu.CompilerParams(use_tc_tiling_on_sc=False),
)
def _sc_gather(table_hbm, idx_hbm, out_hbm, idx_v, rows_v, sem):
    wid = lax.axis_index("s") * _NC + lax.axis_index("c")
    pltpu.sync_copy(idx_hbm.at[wid], idx_v)
    copies = []
    for j in range(_N_CHUNKS):
        copies.append(
            pltpu.async_copy(
                table_hbm.at[idx_v.at[j]],
                rows_v.at[pl.ds(j * _CHUNK, _CHUNK)],
                sem,
            )
        )
    for c in copies:
        c.wait()
    pltpu.sync_copy(rows_v, out_hbm.at[pl.ds(wid * _B_PER_W, _B_PER_W)])


def kernel(terms, go_table):
    tpad = jnp.pad(go_table, ((0, 0), (0, _PAD_DIM - _EMB_DIM)))
    idx = terms.astype(jnp.int32).reshape(_NW, _N_CHUNKS, _CHUNK)
    out = _sc_gather(tpad, idx)
    return out[:, :_EMB_DIM]
